# baseline (speedup 1.0000x reference)
import functools

import jax
import jax.numpy as jnp
from jax import lax
from jax.experimental import pallas as pl
from jax.experimental.pallas import tpu as pltpu
from jax.experimental.pallas import tpu_sc as plsc

EMBEDDING_DIM = 32
BATCH = 16384
HIST_LEN = 200

NUM_CORES = 2
NUM_SUBCORES = 16
NUM_WORKERS = NUM_CORES * NUM_SUBCORES

T1 = HIST_LEN // 8
B1 = BATCH // 128
B1_PER_W = B1 // NUM_WORKERS
N_GROUPS = T1 * 2

_mesh = plsc.VectorSubcoreMesh(core_axis_name="c", subcore_axis_name="s")

_scratch = [
    pltpu.VMEM((2, 1024), jnp.int32),
    pltpu.VMEM((2, 1024, EMBEDDING_DIM), jnp.float32),
    pltpu.VMEM((2, 4, 8, 128), jnp.float32),
]
_scratch += [pltpu.SemaphoreType.DMA] * 6


@functools.partial(
    pl.kernel,
    mesh=_mesh,
    out_type=jax.ShapeDtypeStruct((HIST_LEN, 4, B1, 8, 128), jnp.float32),
    scratch_types=_scratch,
    compiler_params=pltpu.CompilerParams(use_tc_tiling_on_sc=False,
                                         needs_layout_passes=False),
)
def _gather_kernel(tok_hbm, table_hbm, out_hbm, idx_v, rows_v, trans_v,
                   *sems):
    idx_sems = sems[0:2]
    gat_sems = sems[2:4]
    out_sems = sems[4:6]

    wid = lax.axis_index("s") * NUM_CORES + lax.axis_index("c")
    b1_base = wid * B1_PER_W
    iota16 = lax.iota(jnp.int32, 16)
    t1_last = jnp.int32(T1 - 1)

    def start_idx(t1, b1, ib):
        pltpu.async_copy(tok_hbm.at[jnp.minimum(t1, t1_last), b1],
                         idx_v.at[ib], idx_sems[ib])

    def wait_idx(ib):
        pltpu.make_async_copy(tok_hbm.at[0, 0], idx_v.at[ib],
                              idx_sems[ib]).wait()

    def start_gather(ib):
        pltpu.async_copy(table_hbm.at[idx_v.at[ib]], rows_v.at[ib],
                         gat_sems[ib])

    def wait_gather(ib):
        pltpu.make_async_copy(table_hbm.at[idx_v.at[ib]], rows_v.at[ib],
                              gat_sems[ib]).wait()

    def start_out(t, b1, tb):
        for c1 in range(4):
            pltpu.async_copy(trans_v.at[tb, c1], out_hbm.at[t, c1, b1],
                             out_sems[tb])

    def wait_out(tb):
        for c1 in range(4):
            pltpu.make_async_copy(trans_v.at[tb, c1],
                                  out_hbm.at[0, c1, 0], out_sems[tb]).wait()

    def unit_coords(g, i01):
        t1 = lax.shift_right_logical(g, 1)
        b1o = lax.mul(lax.bitwise_and(g, 1), 2) + i01
        return t1, b1_base + b1o

    def transpose_unit(t1, b1, p):
        def kk_body(kk, carry):
            for d in range(2):
                t0 = 2 * kk + d
                wait_out(d)
                rowb = t0 * 128
                ridx = [rowb + b0c * 16 + iota16 for b0c in range(8)]
                for c1 in range(4):
                    for c0 in range(8):
                        cidx = jnp.full((16,), c1 * 8 + c0, jnp.int32)
                        for b0c in range(8):
                            v = plsc.load_gather(rows_v.at[p],
                                                 [ridx[b0c], cidx])
                            trans_v[d, c1, c0, pl.ds(b0c * 16, 16)] = v
                start_out(t1 * 8 + t0, b1, d)
            return carry

        lax.fori_loop(0, 4, kk_body, jnp.int32(0))

    for tb in range(2):
        for c1 in range(4):
            pltpu.async_copy(out_hbm.at[0, c1, 0], trans_v.at[tb, c1],
                             out_sems[tb])

    t1a, b1a = unit_coords(jnp.int32(0), 0)
    start_idx(t1a, b1a, 0)
    t1b, b1b = unit_coords(jnp.int32(0), 1)
    start_idx(t1b, b1b, 1)
    wait_idx(0)
    start_gather(0)

    def group_body(g, carry):
        for i01 in range(2):
            p, p1 = i01, 1 - i01
            t1, b1 = unit_coords(g, i01)
            wait_gather(p)
            wait_idx(p1)
            start_gather(p1)
            t1n, b1n = unit_coords(g + 1, i01)
            start_idx(t1n, b1n, p)
            transpose_unit(t1, b1, p)
        return carry

    lax.fori_loop(0, N_GROUPS, group_body, jnp.int32(0))

    wait_gather(0)
    wait_idx(1)
    wait_out(0)
    wait_out(1)


def kernel(token_ids, embedding_table):
    tok5 = (token_ids.astype(jnp.int32)
            .reshape(128, 128, T1, 8)
            .transpose(2, 0, 3, 1)
            .reshape(T1, B1, 1024))
    out5 = _gather_kernel(tok5, embedding_table)
    return (out5.transpose(2, 4, 0, 1, 3)
            .reshape(BATCH, HIST_LEN, EMBEDDING_DIM))


# speedup vs baseline: 1.2791x; 1.2791x over previous
import functools

import jax
import jax.numpy as jnp
from jax import lax
from jax.experimental import pallas as pl
from jax.experimental.pallas import tpu as pltpu
from jax.experimental.pallas import tpu_sc as plsc

EMBEDDING_DIM = 32
BATCH = 16384
HIST_LEN = 200

NUM_CORES = 2
NUM_SUBCORES = 16
NUM_WORKERS = NUM_CORES * NUM_SUBCORES

T1 = HIST_LEN // 8
B1 = BATCH // 128
B1_PER_W = B1 // NUM_WORKERS
N_GROUPS = T1 * 2

_mesh = plsc.VectorSubcoreMesh(core_axis_name="c", subcore_axis_name="s")

_scratch = [
    pltpu.VMEM((2, 1024), jnp.int32),
    pltpu.VMEM((2, 1024, EMBEDDING_DIM), jnp.float32),
    pltpu.VMEM((2, 4096), jnp.float32),
]
_scratch += [pltpu.SemaphoreType.DMA] * 6


@functools.partial(
    pl.kernel,
    mesh=_mesh,
    out_type=jax.ShapeDtypeStruct((HIST_LEN, 4, B1, 1024), jnp.float32),
    scratch_types=_scratch,
    compiler_params=pltpu.CompilerParams(use_tc_tiling_on_sc=False,
                                         needs_layout_passes=False),
)
def _gather_kernel(tok_hbm, table_hbm, out_hbm, idx_v, rows_v, trans_v,
                   *sems):
    idx_sems = sems[0:2]
    gat_sems = sems[2:4]
    out_sems = sems[4:6]

    wid = lax.axis_index("s") * NUM_CORES + lax.axis_index("c")
    b1_base = wid * B1_PER_W
    iota16 = lax.iota(jnp.int32, 16)
    t1_last = jnp.int32(T1 - 1)

    def start_idx(t1, b1, ib):
        pltpu.async_copy(tok_hbm.at[jnp.minimum(t1, t1_last), b1],
                         idx_v.at[ib], idx_sems[ib])

    def wait_idx(ib):
        pltpu.make_async_copy(tok_hbm.at[0, 0], idx_v.at[ib],
                              idx_sems[ib]).wait()

    def start_gather(ib):
        pltpu.async_copy(table_hbm.at[idx_v.at[ib]], rows_v.at[ib],
                         gat_sems[ib])

    def wait_gather(ib):
        pltpu.make_async_copy(table_hbm.at[idx_v.at[ib]], rows_v.at[ib],
                              gat_sems[ib]).wait()

    def start_out(t, b1, tb):
        for c1 in range(4):
            pltpu.async_copy(trans_v.at[tb, pl.ds(c1 * 1024, 1024)],
                             out_hbm.at[t, c1, b1], out_sems[tb])

    def wait_out(tb):
        for c1 in range(4):
            pltpu.make_async_copy(trans_v.at[tb, pl.ds(c1 * 1024, 1024)],
                                  out_hbm.at[0, c1, 0], out_sems[tb]).wait()

    def unit_coords(g, i01):
        t1 = lax.shift_right_logical(g, 1)
        b1o = lax.mul(lax.bitwise_and(g, 1), 2) + i01
        return t1, b1_base + b1o

    def transpose_unit(t1, b1, p):
        zeros16 = jnp.zeros((16,), jnp.int32)

        def kk_body(kk, carry):
            for d in range(2):
                t0 = 2 * kk + d
                wait_out(d)
                rowb = t0 * 128

                @plsc.parallel_loop(0, 256, unroll=8)
                def _transpose(m):
                    b0c = lax.bitwise_and(m, 7)
                    col = lax.shift_right_logical(m, 3)
                    ridx = rowb + b0c * 16 + iota16
                    cidx = zeros16 + col
                    v = plsc.load_gather(rows_v.at[p], [ridx, cidx])
                    trans_v[d, pl.ds(m * 16, 16)] = v

                start_out(t1 * 8 + t0, b1, d)
            return carry

        lax.fori_loop(0, 4, kk_body, jnp.int32(0))

    for tb in range(2):
        for c1 in range(4):
            pltpu.async_copy(out_hbm.at[0, c1, 0],
                             trans_v.at[tb, pl.ds(c1 * 1024, 1024)],
                             out_sems[tb])

    t1a, b1a = unit_coords(jnp.int32(0), 0)
    start_idx(t1a, b1a, 0)
    t1b, b1b = unit_coords(jnp.int32(0), 1)
    start_idx(t1b, b1b, 1)
    wait_idx(0)
    start_gather(0)

    def group_body(g, carry):
        for i01 in range(2):
            p, p1 = i01, 1 - i01
            t1, b1 = unit_coords(g, i01)
            wait_gather(p)
            wait_idx(p1)
            start_gather(p1)
            t1n, b1n = unit_coords(g + 1, i01)
            start_idx(t1n, b1n, p)
            transpose_unit(t1, b1, p)
        return carry

    lax.fori_loop(0, N_GROUPS, group_body, jnp.int32(0))

    wait_gather(0)
    wait_idx(1)
    wait_out(0)
    wait_out(1)


def kernel(token_ids, embedding_table):
    tok5 = (token_ids.astype(jnp.int32)
            .reshape(128, 128, T1, 8)
            .transpose(2, 0, 3, 1)
            .reshape(T1, B1, 1024))
    out6 = _gather_kernel(tok5, embedding_table)
    return (out6.reshape(HIST_LEN, 4, B1, 8, 128)
            .transpose(2, 4, 0, 1, 3)
            .reshape(BATCH, HIST_LEN, EMBEDDING_DIM))


# speedup vs baseline: 1.2889x; 1.0077x over previous
import functools

import jax
import jax.numpy as jnp
from jax import lax
from jax.experimental import pallas as pl
from jax.experimental.pallas import tpu as pltpu
from jax.experimental.pallas import tpu_sc as plsc

EMBEDDING_DIM = 32
BATCH = 16384
HIST_LEN = 200

NUM_CORES = 2
NUM_SUBCORES = 16
NUM_WORKERS = NUM_CORES * NUM_SUBCORES

T1 = HIST_LEN // 8
B1 = BATCH // 128
B1_PER_W = B1 // NUM_WORKERS
N_GROUPS = T1 * 2

_mesh = plsc.VectorSubcoreMesh(core_axis_name="c", subcore_axis_name="s")

_scratch = [
    pltpu.VMEM((2, 1024), jnp.int32),
    pltpu.VMEM((2, 1024, EMBEDDING_DIM), jnp.float32),
    pltpu.VMEM((4, 4, 1024), jnp.float32),
]
_scratch += [pltpu.SemaphoreType.DMA] * 8


@functools.partial(
    pl.kernel,
    mesh=_mesh,
    out_type=jax.ShapeDtypeStruct((HIST_LEN, 4, B1, 1024), jnp.float32),
    scratch_types=_scratch,
    compiler_params=pltpu.CompilerParams(use_tc_tiling_on_sc=False,
                                         needs_layout_passes=False),
)
def _gather_kernel(tok_hbm, table_hbm, out_hbm, idx_v, rows_v, trans_v,
                   *sems):
    idx_sems = sems[0:2]
    gat_sems = sems[2:4]
    out_sems = sems[4:8]

    wid = lax.axis_index("s") * NUM_CORES + lax.axis_index("c")
    b1_base = wid * B1_PER_W
    iota16 = lax.iota(jnp.int32, 16)
    t1_last = jnp.int32(T1 - 1)

    def start_idx(t1, b1, ib):
        pltpu.async_copy(tok_hbm.at[jnp.minimum(t1, t1_last), b1],
                         idx_v.at[ib], idx_sems[ib])

    def wait_idx(ib):
        pltpu.make_async_copy(tok_hbm.at[0, 0], idx_v.at[ib],
                              idx_sems[ib]).wait()

    def start_gather(ib):
        pltpu.async_copy(table_hbm.at[idx_v.at[ib]], rows_v.at[ib],
                         gat_sems[ib])

    def wait_gather(ib):
        pltpu.make_async_copy(table_hbm.at[idx_v.at[ib]], rows_v.at[ib],
                              gat_sems[ib]).wait()

    def start_out(t, b1, tb):
        pltpu.async_copy(trans_v.at[tb], out_hbm.at[t, :, b1], out_sems[tb])

    def wait_out(tb):
        pltpu.make_async_copy(trans_v.at[tb], out_hbm.at[0, :, 0],
                              out_sems[tb]).wait()

    def unit_coords(g, i01):
        t1 = lax.shift_right_logical(g, 1)
        b1o = lax.mul(lax.bitwise_and(g, 1), 2) + i01
        return t1, b1_base + b1o

    def transpose_unit(t1, b1, p):
        zeros16 = jnp.zeros((16,), jnp.int32)

        def kk_body(kk, carry):
            for d in range(4):
                t0 = 4 * kk + d
                wait_out(d)
                rowb = t0 * 128

                @plsc.parallel_loop(0, 256, unroll=8)
                def _transpose(m):
                    b0c = lax.bitwise_and(m, 7)
                    col = lax.shift_right_logical(m, 3)
                    ridx = rowb + b0c * 16 + iota16
                    cidx = zeros16 + col
                    v = plsc.load_gather(rows_v.at[p], [ridx, cidx])
                    trans_v[d, lax.shift_right_logical(m, 6),
                            pl.ds(lax.bitwise_and(m, 63) * 16, 16)] = v

                start_out(t1 * 8 + t0, b1, d)
            return carry

        lax.fori_loop(0, 2, kk_body, jnp.int32(0))

    for tb in range(4):
        pltpu.async_copy(out_hbm.at[0, :, 0], trans_v.at[tb], out_sems[tb])

    t1a, b1a = unit_coords(jnp.int32(0), 0)
    start_idx(t1a, b1a, 0)
    t1b, b1b = unit_coords(jnp.int32(0), 1)
    start_idx(t1b, b1b, 1)
    wait_idx(0)
    start_gather(0)

    def group_body(g, carry):
        for i01 in range(2):
            p, p1 = i01, 1 - i01
            t1, b1 = unit_coords(g, i01)
            wait_gather(p)
            wait_idx(p1)
            start_gather(p1)
            t1n, b1n = unit_coords(g + 1, i01)
            start_idx(t1n, b1n, p)
            transpose_unit(t1, b1, p)
        return carry

    lax.fori_loop(0, N_GROUPS, group_body, jnp.int32(0))

    wait_gather(0)
    wait_idx(1)
    for tb in range(4):
        wait_out(tb)


def kernel(token_ids, embedding_table):
    tok5 = (token_ids.astype(jnp.int32)
            .reshape(128, 128, T1, 8)
            .transpose(2, 0, 3, 1)
            .reshape(T1, B1, 1024))
    out6 = _gather_kernel(tok5, embedding_table)
    return (out6.reshape(HIST_LEN, 4, B1, 8, 128)
            .transpose(2, 4, 0, 1, 3)
            .reshape(BATCH, HIST_LEN, EMBEDDING_DIM))


# speedup vs baseline: 3.7626x; 2.9191x over previous
import functools

import jax
import jax.numpy as jnp
from jax import lax
from jax.experimental import pallas as pl
from jax.experimental.pallas import tpu as pltpu
from jax.experimental.pallas import tpu_sc as plsc

EMBEDDING_DIM = 32
BATCH = 16384
HIST_LEN = 200

NUM_CORES = 2
NUM_SUBCORES = 16
NUM_WORKERS = NUM_CORES * NUM_SUBCORES

T1 = HIST_LEN // 8
B1 = BATCH // 128
B1_PER_W = B1 // NUM_WORKERS
N_GROUPS = T1 * 2

_mesh = plsc.VectorSubcoreMesh(core_axis_name="c", subcore_axis_name="s")

_scratch = [
    pltpu.VMEM((2, 1024), jnp.int32),
    pltpu.VMEM((2, 1024, EMBEDDING_DIM), jnp.float32),
    pltpu.VMEM((4, 32, 129), jnp.float32),
]
_scratch += [pltpu.SemaphoreType.DMA] * 8


@functools.partial(
    pl.kernel,
    mesh=_mesh,
    out_type=jax.ShapeDtypeStruct((HIST_LEN, 4, B1, 8, 128), jnp.float32),
    scratch_types=_scratch,
    compiler_params=pltpu.CompilerParams(use_tc_tiling_on_sc=False,
                                         needs_layout_passes=False),
)
def _gather_kernel(tok_hbm, table_hbm, out_hbm, idx_v, rows_v, trans_v,
                   *sems):
    idx_sems = sems[0:2]
    gat_sems = sems[2:4]
    out_sems = sems[4:8]

    wid = lax.axis_index("s") * NUM_CORES + lax.axis_index("c")
    b1_base = wid * B1_PER_W
    iota16 = lax.iota(jnp.int32, 16)
    t1_last = jnp.int32(T1 - 1)

    def start_idx(t1, b1, ib):
        pltpu.async_copy(tok_hbm.at[jnp.minimum(t1, t1_last), b1],
                         idx_v.at[ib], idx_sems[ib])

    def wait_idx(ib):
        pltpu.make_async_copy(tok_hbm.at[0, 0], idx_v.at[ib],
                              idx_sems[ib]).wait()

    def start_gather(ib):
        pltpu.async_copy(table_hbm.at[idx_v.at[ib]], rows_v.at[ib],
                         gat_sems[ib])

    def wait_gather(ib):
        pltpu.make_async_copy(table_hbm.at[idx_v.at[ib]], rows_v.at[ib],
                              gat_sems[ib]).wait()

    def start_out(t, b1, tb):
        for c1 in range(4):
            pltpu.async_copy(
                trans_v.at[tb, pl.ds(c1 * 8, 8), pl.ds(0, 128)],
                out_hbm.at[t, c1, b1], out_sems[tb])

    def wait_out(tb):
        for c1 in range(4):
            pltpu.make_async_copy(
                trans_v.at[tb, pl.ds(c1 * 8, 8), pl.ds(0, 128)],
                out_hbm.at[0, c1, 0], out_sems[tb]).wait()

    def unit_coords(g, i01):
        t1 = lax.shift_right_logical(g, 1)
        b1o = lax.mul(lax.bitwise_and(g, 1), 2) + i01
        return t1, b1_base + b1o

    def transpose_unit(t1, b1, p):
        zeros16 = jnp.zeros((16,), jnp.int32)
        row_lo = iota16
        row_hi = iota16 + 16

        def kk_body(kk, carry):
            for d in range(4):
                t0 = 4 * kk + d
                wait_out(d)
                rowb = t0 * 128

                @plsc.parallel_loop(0, 128, unroll=8)
                def _transpose(b0):
                    jj = rowb + b0
                    colv = zeros16 + b0
                    v0 = rows_v[p, jj, pl.ds(0, 16)]
                    v1 = rows_v[p, jj, pl.ds(16, 16)]
                    plsc.store_scatter(trans_v.at[d], [row_lo, colv], v0)
                    plsc.store_scatter(trans_v.at[d], [row_hi, colv], v1)

                start_out(t1 * 8 + t0, b1, d)
            return carry

        lax.fori_loop(0, 2, kk_body, jnp.int32(0))

    for tb in range(4):
        for c1 in range(4):
            pltpu.async_copy(out_hbm.at[0, c1, 0],
                             trans_v.at[tb, pl.ds(c1 * 8, 8), pl.ds(0, 128)],
                             out_sems[tb])

    t1a, b1a = unit_coords(jnp.int32(0), 0)
    start_idx(t1a, b1a, 0)
    t1b, b1b = unit_coords(jnp.int32(0), 1)
    start_idx(t1b, b1b, 1)
    wait_idx(0)
    start_gather(0)

    def group_body(g, carry):
        for i01 in range(2):
            p, p1 = i01, 1 - i01
            t1, b1 = unit_coords(g, i01)
            wait_gather(p)
            wait_idx(p1)
            start_gather(p1)
            t1n, b1n = unit_coords(g + 1, i01)
            start_idx(t1n, b1n, p)
            transpose_unit(t1, b1, p)
        return carry

    lax.fori_loop(0, N_GROUPS, group_body, jnp.int32(0))

    wait_gather(0)
    wait_idx(1)
    for tb in range(4):
        wait_out(tb)


def kernel(token_ids, embedding_table):
    tok5 = (token_ids.astype(jnp.int32)
            .reshape(128, 128, T1, 8)
            .transpose(2, 0, 3, 1)
            .reshape(T1, B1, 1024))
    out5 = _gather_kernel(tok5, embedding_table)
    return (out5.transpose(2, 4, 0, 1, 3)
            .reshape(BATCH, HIST_LEN, EMBEDDING_DIM))
